# baseline (speedup 1.0000x reference)
import functools

import jax
import jax.numpy as jnp
from jax.experimental import pallas as pl
from jax.experimental.pallas import tpu as pltpu


def _upsample_nn(r, fh, fw):
    Hc, Wc, C = r.shape
    r = jnp.broadcast_to(r[:, None, :, :], (Hc, fh, Wc, C))
    r = r.reshape(Hc * fh, Wc, C)
    r = jnp.broadcast_to(r[:, :, None, :], (Hc * fh, Wc, fw, C))
    return r.reshape(Hc * fh, Wc * fw, C)


def _conv3x3(x, w3, b):
    H, W, C = x.shape
    Co = w3.shape[-1]
    M = H * W
    xf = x.reshape(M, C)
    z1 = jnp.zeros((1, C), x.dtype)
    xm = jnp.concatenate([z1, xf[: M - 1]], axis=0)
    xp = jnp.concatenate([xf[1:], z1], axis=0)
    ii = jax.lax.broadcasted_iota(jnp.int32, (M, 1), 0)
    zero = jnp.zeros((), x.dtype)
    xm = jnp.where(ii % W == 0, zero, xm)
    xp = jnp.where(ii % W == W - 1, zero, xp)
    ys = []
    for dy in range(3):
        wdy = w3[dy]
        y = jnp.dot(xm, wdy[:C], preferred_element_type=jnp.float32)
        y = y + jnp.dot(xf, wdy[C : 2 * C], preferred_element_type=jnp.float32)
        y = y + jnp.dot(xp, wdy[2 * C :], preferred_element_type=jnp.float32)
        ys.append(y)
    zW = jnp.zeros((W, Co), jnp.float32)
    acc = ys[1] + b
    acc = acc + jnp.concatenate([zW, ys[0][: M - W]], axis=0)
    acc = acc + jnp.concatenate([ys[2][W:], zW], axis=0)
    return acc.reshape(H, W, Co)


def _fused_kernel(c3_ref, c4_ref, c5_ref,
                  w51_ref, b5_ref, w52_ref, b52_ref,
                  w41_ref, b4_ref, w42_ref, b42_ref,
                  w31_ref, b3_ref, w32_ref, b32_ref,
                  o3_ref, o4_ref, o5_ref):
    H5, W5, C5c = c5_ref.shape[1:]
    H4, W4, C4c = c4_ref.shape[1:]
    H3, W3, C3c = c3_ref.shape[1:]
    Ch = w51_ref.shape[1]

    bf = jnp.bfloat16
    w51 = w51_ref[...].astype(bf)
    w41 = w41_ref[...].astype(bf)
    w31 = w31_ref[...].astype(bf)
    w52 = w52_ref[...].astype(bf)
    w42 = w42_ref[...].astype(bf)
    w32 = w32_ref[...].astype(bf)

    x5 = c5_ref[0].reshape(H5 * W5, C5c).astype(bf)
    y5 = jnp.dot(x5, w51, preferred_element_type=jnp.float32)
    p5x = (y5 + b5_ref[...]).astype(bf).reshape(H5, W5, Ch)
    o5_ref[...] = _conv3x3(p5x, w52, b52_ref[...])[None]

    x4 = c4_ref[0].reshape(H4 * W4, C4c).astype(bf)
    y4 = jnp.dot(x4, w41, preferred_element_type=jnp.float32)
    y4 = (y4 + b4_ref[...]).reshape(H4, W4, Ch)
    r4 = _upsample_nn(p5x.astype(jnp.float32), H4 // H5, W4 // W5)
    p4x = (y4 + r4).astype(bf)
    o4_ref[...] = _conv3x3(p4x, w42, b42_ref[...])[None]

    x3 = c3_ref[0].reshape(H3 * W3, C3c).astype(bf)
    y3 = jnp.dot(x3, w31, preferred_element_type=jnp.float32)
    y3 = (y3 + b3_ref[...]).reshape(H3, W3, Ch)
    r3 = _upsample_nn(p4x.astype(jnp.float32), H3 // H4, W3 // W4)
    p3x = (y3 + r3).astype(bf)
    o3_ref[...] = _conv3x3(p3x, w32, b32_ref[...])[None]


def kernel(C3, C4, C5, p5_1_w, p5_1_b, p5_2_w, p5_2_b,
           p4_1_w, p4_1_b, p4_2_w, p4_2_b,
           p3_1_w, p3_1_b, p3_2_w, p3_2_b):
    N, H3, W3, C3c = C3.shape
    _, H4, W4, C4c = C4.shape
    _, H5, W5, C5c = C5.shape
    Ch = p5_1_w.shape[1]
    Co = p5_2_w.shape[-1]

    w52 = p5_2_w.reshape(3, 3 * Ch, Co)
    w42 = p4_2_w.reshape(3, 3 * Ch, Co)
    w32 = p3_2_w.reshape(3, 3 * Ch, Co)
    b5 = p5_1_b.reshape(1, Ch)
    b4 = p4_1_b.reshape(1, Ch)
    b3 = p3_1_b.reshape(1, Ch)
    b52 = p5_2_b.reshape(1, Co)
    b42 = p4_2_b.reshape(1, Co)
    b32 = p3_2_b.reshape(1, Co)

    res = lambda *blk: pl.BlockSpec(blk, lambda n: (0,) * len(blk))
    out3, out4, out5 = pl.pallas_call(
        _fused_kernel,
        out_shape=(
            jax.ShapeDtypeStruct((N, H3, W3, Co), jnp.float32),
            jax.ShapeDtypeStruct((N, H4, W4, Co), jnp.float32),
            jax.ShapeDtypeStruct((N, H5, W5, Co), jnp.float32),
        ),
        grid=(N,),
        in_specs=[
            pl.BlockSpec((1, H3, W3, C3c), lambda n: (n, 0, 0, 0)),
            pl.BlockSpec((1, H4, W4, C4c), lambda n: (n, 0, 0, 0)),
            pl.BlockSpec((1, H5, W5, C5c), lambda n: (n, 0, 0, 0)),
            res(C5c, Ch), res(1, Ch), res(3, 3 * Ch, Co), res(1, Co),
            res(C4c, Ch), res(1, Ch), res(3, 3 * Ch, Co), res(1, Co),
            res(C3c, Ch), res(1, Ch), res(3, 3 * Ch, Co), res(1, Co),
        ],
        out_specs=(
            pl.BlockSpec((1, H3, W3, Co), lambda n: (n, 0, 0, 0)),
            pl.BlockSpec((1, H4, W4, Co), lambda n: (n, 0, 0, 0)),
            pl.BlockSpec((1, H5, W5, Co), lambda n: (n, 0, 0, 0)),
        ),
        compiler_params=pltpu.CompilerParams(
            dimension_semantics=("parallel",),
            vmem_limit_bytes=100 * 1024 * 1024),
    )(C3, C4, C5,
      p5_1_w, b5, w52, b52,
      p4_1_w, b4, w42, b42,
      p3_1_w, b3, w32, b32)
    return [out3, out4, out5]


# speedup vs baseline: 1.2527x; 1.2527x over previous
import functools

import jax
import jax.numpy as jnp
from jax.experimental import pallas as pl
from jax.experimental.pallas import tpu as pltpu


def _upsample_nn(r, fh, fw):
    Hc, Wc, C = r.shape
    r = jnp.broadcast_to(r[:, None, :, :], (Hc, fh, Wc, C))
    r = r.reshape(Hc * fh, Wc, C)
    r = jnp.broadcast_to(r[:, :, None, :], (Hc * fh, Wc, fw, C))
    return r.reshape(Hc * fh, Wc * fw, C)


def _conv3x3(x, w3, b):
    H, W, C = x.shape
    Co = w3.shape[-1]
    M = H * W
    xf = x.reshape(M, C)
    z1 = jnp.zeros((1, C), x.dtype)
    xm = jnp.concatenate([z1, xf[: M - 1]], axis=0)
    xp = jnp.concatenate([xf[1:], z1], axis=0)
    ii = jax.lax.broadcasted_iota(jnp.int32, (M, 1), 0)
    zero = jnp.zeros((), x.dtype)
    xm = jnp.where(ii % W == 0, zero, xm)
    xp = jnp.where(ii % W == W - 1, zero, xp)
    patch = jnp.concatenate([xm, xf, xp], axis=1)
    y0 = jnp.dot(patch, w3[0], preferred_element_type=jnp.float32)
    y1 = jnp.dot(patch, w3[1], preferred_element_type=jnp.float32)
    y2 = jnp.dot(patch, w3[2], preferred_element_type=jnp.float32)
    zW = jnp.zeros((W, Co), jnp.float32)
    acc = y1 + b
    acc = acc + jnp.concatenate([zW, y0[: M - W]], axis=0)
    acc = acc + jnp.concatenate([y2[W:], zW], axis=0)
    return acc.reshape(H, W, Co)


def _fused_kernel(c3_ref, c4_ref, c5_ref,
                  w51_ref, b5_ref, w52_ref, b52_ref,
                  w41_ref, b4_ref, w42_ref, b42_ref,
                  w31_ref, b3_ref, w32_ref, b32_ref,
                  o3_ref, o4_ref, o5_ref):
    H5, W5, C5c = c5_ref.shape[1:]
    H4, W4, C4c = c4_ref.shape[1:]
    H3, W3, C3c = c3_ref.shape[1:]
    Ch = w51_ref.shape[1]

    bf = jnp.bfloat16
    w51 = w51_ref[...].astype(bf)
    w41 = w41_ref[...].astype(bf)
    w31 = w31_ref[...].astype(bf)
    w52 = w52_ref[...].astype(bf)
    w42 = w42_ref[...].astype(bf)
    w32 = w32_ref[...].astype(bf)

    x5 = c5_ref[0].reshape(H5 * W5, C5c).astype(bf)
    y5 = jnp.dot(x5, w51, preferred_element_type=jnp.float32)
    p5x = (y5 + b5_ref[...]).astype(bf).reshape(H5, W5, Ch)
    o5_ref[...] = _conv3x3(p5x, w52, b52_ref[...])[None]

    x4 = c4_ref[0].reshape(H4 * W4, C4c).astype(bf)
    y4 = jnp.dot(x4, w41, preferred_element_type=jnp.float32)
    y4 = (y4 + b4_ref[...]).reshape(H4, W4, Ch)
    r4 = _upsample_nn(p5x, H4 // H5, W4 // W5)
    p4x = (y4 + r4.astype(jnp.float32)).astype(bf)
    o4_ref[...] = _conv3x3(p4x, w42, b42_ref[...])[None]

    x3 = c3_ref[0].reshape(H3 * W3, C3c).astype(bf)
    y3 = jnp.dot(x3, w31, preferred_element_type=jnp.float32)
    y3 = (y3 + b3_ref[...]).reshape(H3, W3, Ch)
    r3 = _upsample_nn(p4x, H3 // H4, W3 // W4)
    p3x = (y3 + r3.astype(jnp.float32)).astype(bf)
    o3_ref[...] = _conv3x3(p3x, w32, b32_ref[...])[None]


def kernel(C3, C4, C5, p5_1_w, p5_1_b, p5_2_w, p5_2_b,
           p4_1_w, p4_1_b, p4_2_w, p4_2_b,
           p3_1_w, p3_1_b, p3_2_w, p3_2_b):
    N, H3, W3, C3c = C3.shape
    _, H4, W4, C4c = C4.shape
    _, H5, W5, C5c = C5.shape
    Ch = p5_1_w.shape[1]
    Co = p5_2_w.shape[-1]

    w52 = p5_2_w.reshape(3, 3 * Ch, Co)
    w42 = p4_2_w.reshape(3, 3 * Ch, Co)
    w32 = p3_2_w.reshape(3, 3 * Ch, Co)
    b5 = p5_1_b.reshape(1, Ch)
    b4 = p4_1_b.reshape(1, Ch)
    b3 = p3_1_b.reshape(1, Ch)
    b52 = p5_2_b.reshape(1, Co)
    b42 = p4_2_b.reshape(1, Co)
    b32 = p3_2_b.reshape(1, Co)

    res = lambda *blk: pl.BlockSpec(blk, lambda n: (0,) * len(blk))
    out3, out4, out5 = pl.pallas_call(
        _fused_kernel,
        out_shape=(
            jax.ShapeDtypeStruct((N, H3, W3, Co), jnp.float32),
            jax.ShapeDtypeStruct((N, H4, W4, Co), jnp.float32),
            jax.ShapeDtypeStruct((N, H5, W5, Co), jnp.float32),
        ),
        grid=(N,),
        in_specs=[
            pl.BlockSpec((1, H3, W3, C3c), lambda n: (n, 0, 0, 0)),
            pl.BlockSpec((1, H4, W4, C4c), lambda n: (n, 0, 0, 0)),
            pl.BlockSpec((1, H5, W5, C5c), lambda n: (n, 0, 0, 0)),
            res(C5c, Ch), res(1, Ch), res(3, 3 * Ch, Co), res(1, Co),
            res(C4c, Ch), res(1, Ch), res(3, 3 * Ch, Co), res(1, Co),
            res(C3c, Ch), res(1, Ch), res(3, 3 * Ch, Co), res(1, Co),
        ],
        out_specs=(
            pl.BlockSpec((1, H3, W3, Co), lambda n: (n, 0, 0, 0)),
            pl.BlockSpec((1, H4, W4, Co), lambda n: (n, 0, 0, 0)),
            pl.BlockSpec((1, H5, W5, Co), lambda n: (n, 0, 0, 0)),
        ),
        compiler_params=pltpu.CompilerParams(
            dimension_semantics=("parallel",),
            vmem_limit_bytes=100 * 1024 * 1024),
    )(C3, C4, C5,
      p5_1_w, b5, w52, b52,
      p4_1_w, b4, w42, b42,
      p3_1_w, b3, w32, b32)
    return [out3, out4, out5]


# speedup vs baseline: 1.4613x; 1.1665x over previous
import jax
import jax.numpy as jnp
from jax.experimental import pallas as pl
from jax.experimental.pallas import tpu as pltpu


def _upsample_nn(r, fh, fw):
    Hc, Wc, C = r.shape
    r = jnp.broadcast_to(r[:, None, :, :], (Hc, fh, Wc, C))
    r = r.reshape(Hc * fh, Wc, C)
    r = jnp.broadcast_to(r[:, :, None, :], (Hc * fh, Wc, fw, C))
    return r.reshape(Hc * fh, Wc * fw, C)


def _conv3x3(x, w3, b):
    H, W, C = x.shape
    Co = w3.shape[-1]
    zcol = jnp.zeros((H, 1, C), x.dtype)
    p0 = jnp.concatenate([zcol, x[:, : W - 1, :]], axis=1)
    p2 = jnp.concatenate([x[:, 1:, :], zcol], axis=1)
    patch = jnp.concatenate([p0, x, p2], axis=-1).reshape(H * W, 3 * C)
    wcat = jnp.concatenate([w3[0], w3[1], w3[2]], axis=1)
    y = jnp.dot(patch, wcat, preferred_element_type=jnp.float32)
    y0 = y[:, :Co].reshape(H, W, Co)
    y1 = y[:, Co : 2 * Co].reshape(H, W, Co)
    y2 = y[:, 2 * Co :].reshape(H, W, Co)
    zrow = jnp.zeros((1, W, Co), jnp.float32)
    acc = y1 + b.reshape(1, 1, Co)
    acc = acc + jnp.concatenate([zrow, y0[: H - 1]], axis=0)
    acc = acc + jnp.concatenate([y2[1:], zrow], axis=0)
    return acc


def _fused_kernel(c3_ref, c4_ref, c5_ref,
                  w51_ref, b5_ref, w52_ref, b52_ref,
                  w41_ref, b4_ref, w42_ref, b42_ref,
                  w31_ref, b3_ref, w32_ref, b32_ref,
                  o3_ref, o4_ref, o5_ref):
    H5, W5, C5c = c5_ref.shape[1:]
    H4, W4, C4c = c4_ref.shape[1:]
    H3, W3, C3c = c3_ref.shape[1:]
    Ch = w51_ref.shape[1]

    bf = jnp.bfloat16
    w51 = w51_ref[...].astype(bf)
    w41 = w41_ref[...].astype(bf)
    w31 = w31_ref[...].astype(bf)
    w52 = w52_ref[...].astype(bf)
    w42 = w42_ref[...].astype(bf)
    w32 = w32_ref[...].astype(bf)

    x5 = c5_ref[0].reshape(H5 * W5, C5c).astype(bf)
    y5 = jnp.dot(x5, w51, preferred_element_type=jnp.float32)
    p5x = (y5 + b5_ref[...]).astype(bf).reshape(H5, W5, Ch)
    o5_ref[...] = _conv3x3(p5x, w52, b52_ref[...])[None]

    x4 = c4_ref[0].reshape(H4 * W4, C4c).astype(bf)
    y4 = jnp.dot(x4, w41, preferred_element_type=jnp.float32)
    y4 = (y4 + b4_ref[...]).reshape(H4, W4, Ch)
    r4 = _upsample_nn(p5x.astype(jnp.float32), H4 // H5, W4 // W5)
    p4x = (y4 + r4).astype(bf)
    o4_ref[...] = _conv3x3(p4x, w42, b42_ref[...])[None]

    x3 = c3_ref[0].reshape(H3 * W3, C3c).astype(bf)
    y3 = jnp.dot(x3, w31, preferred_element_type=jnp.float32)
    y3 = (y3 + b3_ref[...]).reshape(H3, W3, Ch)
    r3 = _upsample_nn(p4x.astype(jnp.float32), H3 // H4, W3 // W4)
    p3x = (y3 + r3).astype(bf)
    o3_ref[...] = _conv3x3(p3x, w32, b32_ref[...])[None]


def kernel(C3, C4, C5, p5_1_w, p5_1_b, p5_2_w, p5_2_b,
           p4_1_w, p4_1_b, p4_2_w, p4_2_b,
           p3_1_w, p3_1_b, p3_2_w, p3_2_b):
    N, H3, W3, C3c = C3.shape
    _, H4, W4, C4c = C4.shape
    _, H5, W5, C5c = C5.shape
    Ch = p5_1_w.shape[1]
    Co = p5_2_w.shape[-1]

    w52 = p5_2_w.reshape(3, 3 * Ch, Co)
    w42 = p4_2_w.reshape(3, 3 * Ch, Co)
    w32 = p3_2_w.reshape(3, 3 * Ch, Co)
    b5 = p5_1_b.reshape(1, Ch)
    b4 = p4_1_b.reshape(1, Ch)
    b3 = p3_1_b.reshape(1, Ch)
    b52 = p5_2_b.reshape(1, Co)
    b42 = p4_2_b.reshape(1, Co)
    b32 = p3_2_b.reshape(1, Co)

    res = lambda *blk: pl.BlockSpec(blk, lambda n: (0,) * len(blk))
    out3, out4, out5 = pl.pallas_call(
        _fused_kernel,
        out_shape=(
            jax.ShapeDtypeStruct((N, H3, W3, Co), jnp.float32),
            jax.ShapeDtypeStruct((N, H4, W4, Co), jnp.float32),
            jax.ShapeDtypeStruct((N, H5, W5, Co), jnp.float32),
        ),
        grid=(N,),
        in_specs=[
            pl.BlockSpec((1, H3, W3, C3c), lambda n: (n, 0, 0, 0)),
            pl.BlockSpec((1, H4, W4, C4c), lambda n: (n, 0, 0, 0)),
            pl.BlockSpec((1, H5, W5, C5c), lambda n: (n, 0, 0, 0)),
            res(C5c, Ch), res(1, Ch), res(3, 3 * Ch, Co), res(1, Co),
            res(C4c, Ch), res(1, Ch), res(3, 3 * Ch, Co), res(1, Co),
            res(C3c, Ch), res(1, Ch), res(3, 3 * Ch, Co), res(1, Co),
        ],
        out_specs=(
            pl.BlockSpec((1, H3, W3, Co), lambda n: (n, 0, 0, 0)),
            pl.BlockSpec((1, H4, W4, Co), lambda n: (n, 0, 0, 0)),
            pl.BlockSpec((1, H5, W5, Co), lambda n: (n, 0, 0, 0)),
        ),
        compiler_params=pltpu.CompilerParams(
            dimension_semantics=("parallel",),
            vmem_limit_bytes=100 * 1024 * 1024),
    )(C3, C4, C5,
      p5_1_w, b5, w52, b52,
      p4_1_w, b4, w42, b42,
      p3_1_w, b3, w32, b32)
    return [out3, out4, out5]
